# Initial kernel scaffold; baseline (speedup 1.0000x reference)
import jax
import jax.numpy as jnp
from jax.experimental import pallas as pl


def kernel(x, W):
    raise NotImplementedError("write your pallas kernel here")


# speedup vs baseline: 1.3545x; 1.3545x over previous
import functools
import math

import jax
import jax.numpy as jnp
from jax.experimental import pallas as pl
from jax.experimental.pallas import tpu as pltpu

_D_MODEL = 768
_N_EXP = 64
_Z_COEF = 0.001
_AUX_COEF = 0.01
_CAP_FACTOR = 1.0
_MIN_CAP = 4

_TB = 2048


def _router_body(x_ref, wt_ref, idx_ref, prob_ref, cnt_ref, aux_ref,
                 p_acc, z_acc, *, n_tokens):
    i = pl.program_id(0)
    nb = pl.num_programs(0)

    logits = jnp.dot(x_ref[...], wt_ref[...],
                     preferred_element_type=jnp.float32)
    m = jnp.max(logits, axis=1, keepdims=True)
    e = jnp.exp(logits - m)
    s = jnp.sum(e, axis=1, keepdims=True)
    eid = jax.lax.broadcasted_iota(jnp.int32, logits.shape, 1)
    amax = jnp.min(jnp.where(logits >= m, eid, _N_EXP), axis=1)
    idx_ref[...] = amax
    prob_ref[...] = 1.0 / s[:, 0]
    lse = m[:, 0] + jnp.log(s[:, 0])

    onehot = (eid == amax[:, None]).astype(jnp.int32)
    cnt_blk = jnp.sum(onehot, axis=0)
    p_blk = jnp.sum(e * (1.0 / s), axis=0)
    z_blk = jnp.sum(lse * lse)

    @pl.when(i == 0)
    def _init():
        cnt_ref[...] = jnp.zeros_like(cnt_ref)
        p_acc[...] = jnp.zeros_like(p_acc)
        z_acc[0, 0] = 0.0

    cnt_ref[...] += cnt_blk
    p_acc[...] += p_blk
    z_acc[0, 0] += z_blk

    @pl.when(i == nb - 1)
    def _finish():
        counts_f = cnt_ref[...].astype(jnp.float32)
        inv_n = 1.0 / n_tokens
        aux = (_AUX_COEF * _N_EXP * jnp.sum(counts_f * p_acc[...])
               * (inv_n * inv_n) + _Z_COEF * z_acc[0, 0] * inv_n)
        aux_ref[...] = jnp.reshape(aux, (1, 1))


def kernel(x, W):
    B, T, D = x.shape
    n = B * T
    x_flat = x.reshape(n, D)
    wt = W.T
    nb = n // _TB

    body = functools.partial(_router_body, n_tokens=float(n))
    idx, prob, counts, aux = pl.pallas_call(
        body,
        grid=(nb,),
        in_specs=[
            pl.BlockSpec((_TB, D), lambda i: (i, 0)),
            pl.BlockSpec((D, _N_EXP), lambda i: (0, 0)),
        ],
        out_specs=[
            pl.BlockSpec((_TB,), lambda i: (i,)),
            pl.BlockSpec((_TB,), lambda i: (i,)),
            pl.BlockSpec((_N_EXP,), lambda i: (0,)),
            pl.BlockSpec((1, 1), lambda i: (0, 0)),
        ],
        out_shape=[
            jax.ShapeDtypeStruct((n,), jnp.int32),
            jax.ShapeDtypeStruct((n,), jnp.float32),
            jax.ShapeDtypeStruct((_N_EXP,), jnp.int32),
            jax.ShapeDtypeStruct((1, 1), jnp.float32),
        ],
        scratch_shapes=[
            pltpu.VMEM((_N_EXP,), jnp.float32),
            pltpu.SMEM((1, 1), jnp.float32),
        ],
    )(x_flat, wt)

    capacity = max(_MIN_CAP, math.ceil(_CAP_FACTOR * n / _N_EXP))
    return (idx, prob, counts, jnp.array(capacity, dtype=jnp.int32),
            aux[0, 0])
